# baseline (speedup 1.0000x reference)
import jax
import jax.numpy as jnp
from jax.experimental import pallas as pl
from jax.experimental.pallas import tpu as pltpu

_CHUNKS = 4


def _packed_kernel(x0_ref, x1_ref, x2_ref, x3_ref, w1_ref, b1_ref, w2_ref,
                   b2_ref, w3_ref, b3_ref, o_ref):
    x16 = jnp.concatenate(
        [x0_ref[...], x1_ref[...], x2_ref[...], x3_ref[...]], axis=0)
    h = jnp.dot(w1_ref[...], x16, preferred_element_type=jnp.float32)
    h = jnp.maximum(h + b1_ref[...], 0.0)
    h = jnp.dot(w2_ref[...], h.astype(jnp.bfloat16),
                preferred_element_type=jnp.float32)
    h = jnp.maximum(h + b2_ref[...], 0.0)
    z = (jnp.dot(w3_ref[...], h.astype(jnp.bfloat16),
                 preferred_element_type=jnp.float32)
         + b3_ref[...])
    o_ref[...] = 1.0 / (1.0 + jnp.exp(-z))


def _softmax_kernel(xt_ref, w1_ref, b1_ref, w2_ref, b2_ref, w3_ref, b3_ref,
                    o_ref):
    h = jnp.dot(w1_ref[...], xt_ref[...], preferred_element_type=jnp.float32)
    h = jnp.maximum(h + b1_ref[...], 0.0)
    h = jnp.dot(w2_ref[...], h, preferred_element_type=jnp.float32)
    h = jnp.maximum(h + b2_ref[...], 0.0)
    logits = (jnp.dot(w3_ref[...], h, preferred_element_type=jnp.float32)
              + b3_ref[...])
    m = jnp.max(logits, axis=0, keepdims=True)
    e = jnp.exp(logits - m)
    o_ref[...] = (e / jnp.sum(e, axis=0, keepdims=True)).astype(o_ref.dtype)


def _round_up(n, m):
    return ((n + m - 1) // m) * m


def _blockdiag(m, copies):
    eye = jnp.eye(copies, dtype=m.dtype)
    r, c = m.shape
    return jnp.einsum('ij,rc->irjc', eye, m).reshape(copies * r, copies * c)


def _general_forward(x, w1, b1, w2, b2, w3, b3):
    B, F = x.shape
    h1, h2, out_dim = w1.shape[1], w2.shape[1], w3.shape[1]
    tb = 4096
    padded_b = _round_up(B, tb)
    xt = x.T
    if padded_b != B:
        xt = jnp.pad(xt, ((0, 0), (0, padded_b - B)))
    w1t, w2t, w3t = w1.T, w2.T, w3.T
    b1t = b1.reshape(h1, 1)
    b2t = b2.reshape(h2, 1)
    b3t = b3.reshape(out_dim, 1)

    def rep(arr):
        nd = arr.ndim
        return pl.BlockSpec(arr.shape, lambda i, _n=nd: (0,) * _n)

    out_t = pl.pallas_call(
        _softmax_kernel,
        out_shape=jax.ShapeDtypeStruct((out_dim, padded_b), jnp.float32),
        grid_spec=pl.GridSpec(
            grid=(padded_b // tb,),
            in_specs=[
                pl.BlockSpec((F, tb), lambda i: (0, i)),
                rep(w1t), rep(b1t), rep(w2t), rep(b2t), rep(w3t), rep(b3t),
            ],
            out_specs=pl.BlockSpec((out_dim, tb), lambda i: (0, i)),
        ),
        compiler_params=pltpu.CompilerParams(
            dimension_semantics=("parallel",),
        ),
    )(xt, w1t, b1t, w2t, b2t, w3t, b3t)
    return out_t[:, :B].T


def kernel(x, w1, b1, w2, b2, w3, b3):
    B, F = x.shape
    h1 = w1.shape[1]
    h2 = w2.shape[1]
    out_dim = w3.shape[1]

    if out_dim != 2 or F != 4 or h1 != 32 or h2 != 32:
        return _general_forward(x, w1, b1, w2, b2, w3, b3)

    tb = 4096
    padded_b = _round_up(B, _CHUNKS * tb) if B % (_CHUNKS * tb) else B
    xt = x.astype(jnp.bfloat16).T
    if padded_b != B:
        xt = jnp.pad(xt, ((0, 0), (0, padded_b - B)))
    bq = padded_b // _CHUNKS
    steps = bq // tb

    w1b = _blockdiag(w1.T, _CHUNKS).astype(jnp.bfloat16)
    w2b = _blockdiag(w2.T, _CHUNKS).astype(jnp.bfloat16)
    w3d = jnp.stack([w3[:, 0] - w3[:, 1], w3[:, 1] - w3[:, 0]], axis=1)
    w3b = _blockdiag(w3d.T, _CHUNKS).astype(jnp.bfloat16)
    b1b = jnp.tile(b1.reshape(h1, 1), (_CHUNKS, 1))
    b2b = jnp.tile(b2.reshape(h2, 1), (_CHUNKS, 1))
    b3d = jnp.stack([b3[0, 0] - b3[0, 1], b3[0, 1] - b3[0, 0]])
    b3b = jnp.tile(b3d.reshape(2, 1), (_CHUNKS, 1))

    def rep(arr):
        nd = arr.ndim
        return pl.BlockSpec(arr.shape, lambda i, _n=nd: (0,) * _n)

    def chunk_spec(c):
        return pl.BlockSpec((F, tb), lambda i, _c=c: (0, _c * steps + i))

    flops = 2 * padded_b * (F * h1 + h1 * h2 + h2 * 2)
    bytes_accessed = 4 * (padded_b * (F + 2) + w1b.size + w2b.size
                          + w3b.size + 3 * 128)

    out_p = pl.pallas_call(
        _packed_kernel,
        out_shape=jax.ShapeDtypeStruct((2 * _CHUNKS, bq), jnp.float32),
        grid_spec=pl.GridSpec(
            grid=(steps,),
            in_specs=[
                chunk_spec(0), chunk_spec(1), chunk_spec(2), chunk_spec(3),
                rep(w1b), rep(b1b), rep(w2b), rep(b2b), rep(w3b), rep(b3b),
            ],
            out_specs=pl.BlockSpec((2 * _CHUNKS, tb), lambda i: (0, i)),
        ),
        compiler_params=pltpu.CompilerParams(
            dimension_semantics=("parallel",),
        ),
        cost_estimate=pl.CostEstimate(
            flops=flops,
            bytes_accessed=bytes_accessed,
            transcendentals=padded_b * 2,
        ),
    )(xt, xt, xt, xt, w1b, b1b, w2b, b2b, w3b, b3b)

    out = out_p.reshape(_CHUNKS, 2, bq).transpose(0, 2, 1).reshape(padded_b, 2)
    if padded_b != B:
        out = out[:B]
    return out


# speedup vs baseline: 1.0887x; 1.0887x over previous
import jax
import jax.numpy as jnp
from jax.experimental import pallas as pl
from jax.experimental.pallas import tpu as pltpu

_CHUNKS = 4


def _packed_kernel(x0_ref, x1_ref, x2_ref, x3_ref, w1_ref, b1_ref, w2_ref,
                   b2_ref, w3_ref, b3_ref, o_ref):
    x16 = jnp.concatenate(
        [x0_ref[...], x1_ref[...], x2_ref[...], x3_ref[...]],
        axis=0).astype(jnp.bfloat16)
    h = jnp.dot(w1_ref[...], x16, preferred_element_type=jnp.float32)
    h = jnp.maximum(h + b1_ref[...], 0.0)
    h = jnp.dot(w2_ref[...], h.astype(jnp.bfloat16),
                preferred_element_type=jnp.float32)
    h = jnp.maximum(h + b2_ref[...], 0.0)
    z = (jnp.dot(w3_ref[...], h.astype(jnp.bfloat16),
                 preferred_element_type=jnp.float32)
         + b3_ref[...])
    o_ref[...] = 1.0 / (1.0 + jnp.exp(-z))


def _softmax_kernel(xt_ref, w1_ref, b1_ref, w2_ref, b2_ref, w3_ref, b3_ref,
                    o_ref):
    h = jnp.dot(w1_ref[...], xt_ref[...], preferred_element_type=jnp.float32)
    h = jnp.maximum(h + b1_ref[...], 0.0)
    h = jnp.dot(w2_ref[...], h, preferred_element_type=jnp.float32)
    h = jnp.maximum(h + b2_ref[...], 0.0)
    logits = (jnp.dot(w3_ref[...], h, preferred_element_type=jnp.float32)
              + b3_ref[...])
    m = jnp.max(logits, axis=0, keepdims=True)
    e = jnp.exp(logits - m)
    o_ref[...] = (e / jnp.sum(e, axis=0, keepdims=True)).astype(o_ref.dtype)


def _round_up(n, m):
    return ((n + m - 1) // m) * m


def _blockdiag(m, copies):
    eye = jnp.eye(copies, dtype=m.dtype)
    r, c = m.shape
    return jnp.einsum('ij,rc->irjc', eye, m).reshape(copies * r, copies * c)


def _general_forward(x, w1, b1, w2, b2, w3, b3):
    B, F = x.shape
    h1, h2, out_dim = w1.shape[1], w2.shape[1], w3.shape[1]
    tb = 4096
    padded_b = _round_up(B, tb)
    xt = x.T
    if padded_b != B:
        xt = jnp.pad(xt, ((0, 0), (0, padded_b - B)))
    w1t, w2t, w3t = w1.T, w2.T, w3.T
    b1t = b1.reshape(h1, 1)
    b2t = b2.reshape(h2, 1)
    b3t = b3.reshape(out_dim, 1)

    def rep(arr):
        nd = arr.ndim
        return pl.BlockSpec(arr.shape, lambda i, _n=nd: (0,) * _n)

    out_t = pl.pallas_call(
        _softmax_kernel,
        out_shape=jax.ShapeDtypeStruct((out_dim, padded_b), jnp.float32),
        grid_spec=pl.GridSpec(
            grid=(padded_b // tb,),
            in_specs=[
                pl.BlockSpec((F, tb), lambda i: (0, i)),
                rep(w1t), rep(b1t), rep(w2t), rep(b2t), rep(w3t), rep(b3t),
            ],
            out_specs=pl.BlockSpec((out_dim, tb), lambda i: (0, i)),
        ),
        compiler_params=pltpu.CompilerParams(
            dimension_semantics=("parallel",),
        ),
    )(xt, w1t, b1t, w2t, b2t, w3t, b3t)
    return out_t[:, :B].T


def kernel(x, w1, b1, w2, b2, w3, b3):
    B, F = x.shape
    h1 = w1.shape[1]
    h2 = w2.shape[1]
    out_dim = w3.shape[1]

    if out_dim != 2 or F != 4 or h1 != 32 or h2 != 32:
        return _general_forward(x, w1, b1, w2, b2, w3, b3)

    tb = 4096
    padded_b = _round_up(B, _CHUNKS * tb) if B % (_CHUNKS * tb) else B
    xt = x.T
    if padded_b != B:
        xt = jnp.pad(xt, ((0, 0), (0, padded_b - B)))
    bq = padded_b // _CHUNKS
    steps = bq // tb

    w1b = _blockdiag(w1.T, _CHUNKS).astype(jnp.bfloat16)
    w2b = _blockdiag(w2.T, _CHUNKS).astype(jnp.bfloat16)
    w3d = jnp.stack([w3[:, 0] - w3[:, 1], w3[:, 1] - w3[:, 0]], axis=1)
    w3b = _blockdiag(w3d.T, _CHUNKS).astype(jnp.bfloat16)
    b1b = jnp.tile(b1.reshape(h1, 1), (_CHUNKS, 1))
    b2b = jnp.tile(b2.reshape(h2, 1), (_CHUNKS, 1))
    b3d = jnp.stack([b3[0, 0] - b3[0, 1], b3[0, 1] - b3[0, 0]])
    b3b = jnp.tile(b3d.reshape(2, 1), (_CHUNKS, 1))

    def rep(arr):
        nd = arr.ndim
        return pl.BlockSpec(arr.shape, lambda i, _n=nd: (0,) * _n)

    def chunk_spec(c):
        return pl.BlockSpec((F, tb), lambda i, _c=c: (0, _c * steps + i))

    flops = 2 * padded_b * (F * h1 + h1 * h2 + h2 * 2)
    bytes_accessed = 4 * (padded_b * (F + 2) + w1b.size + w2b.size
                          + w3b.size + 3 * 128)

    out_p = pl.pallas_call(
        _packed_kernel,
        out_shape=jax.ShapeDtypeStruct((2 * _CHUNKS, bq), jnp.float32),
        grid_spec=pl.GridSpec(
            grid=(steps,),
            in_specs=[
                chunk_spec(0), chunk_spec(1), chunk_spec(2), chunk_spec(3),
                rep(w1b), rep(b1b), rep(w2b), rep(b2b), rep(w3b), rep(b3b),
            ],
            out_specs=pl.BlockSpec((2 * _CHUNKS, tb), lambda i: (0, i)),
        ),
        compiler_params=pltpu.CompilerParams(
            dimension_semantics=("parallel",),
        ),
        cost_estimate=pl.CostEstimate(
            flops=flops,
            bytes_accessed=bytes_accessed,
            transcendentals=padded_b * 2,
        ),
    )(xt, xt, xt, xt, w1b, b1b, w2b, b2b, w3b, b3b)

    out = out_p.reshape(_CHUNKS, 2, bq).transpose(0, 2, 1).reshape(padded_b, 2)
    if padded_b != B:
        out = out[:B]
    return out
